# baseline (speedup 1.0000x reference)
import jax
import jax.numpy as jnp
from jax import lax
from jax.experimental import pallas as pl
from jax.experimental.pallas import tpu as pltpu
from jax.experimental.pallas import tpu_sc as plsc

N = 16384
D = 256
B = 16
L = 16
NS = 16
CHUNK = 128
ROWS_PER_W = N // NS
NCHUNK = ROWS_PER_W // CHUNK
GRP = CHUNK // L
DL_PER_W = D // NS
DJ = D // L

f32 = jnp.float32
i32 = jnp.int32


def _sigmoid(v):
    e = jnp.exp(-jnp.abs(v))
    return jnp.where(v >= 0.0, 1.0 / (1.0 + e), e / (1.0 + e))


def _tanh(v):
    e = jnp.exp(-2.0 * jnp.abs(v))
    return jnp.sign(v) * (1.0 - e) / (1.0 + e)


def _body(x_hbm, batch_hbm, wt_hbm, out_hbm,
          xbuf, bidx_all, sums_l, out_l, cnt_l, slabv, redrow, cslabv,
          sums_v, meanT, wtv, trows, ttw,
          sh_slab, sh_cslab, sh_sums):
    cid = lax.axis_index("c")
    sid = lax.axis_index("s")
    base = sid * ROWS_PER_W
    iota = lax.iota(i32, L)
    zeros16 = jnp.zeros((L,), f32)

    def zr(r, _):
        def zc(j, _):
            sums_l[r, pl.ds(j * L, L)] = zeros16
            out_l[r, pl.ds(j * L, L)] = zeros16
            return 0
        lax.fori_loop(0, DJ, zc, 0)
        return 0
    lax.fori_loop(0, B, zr, 0)

    pltpu.sync_copy(batch_hbm.at[pl.ds(base, ROWS_PER_W)], bidx_all)

    def p1_chunk(c, cnt_acc):
        pltpu.sync_copy(x_hbm.at[pl.ds(base + c * CHUNK, CHUNK)], xbuf)

        def grp(g, cnt_in):
            bv = bidx_all[pl.ds(c * CHUNK + g * L, L)]
            mn = lax.reduce_min(bv, (0,))
            mx = lax.reduce_max(bv, (0,))

            def uniform(cnt):
                def col(j, _):
                    s = zeros16
                    for r in range(L):
                        s = s + xbuf[g * L + r, pl.ds(j * L, L)]
                    sums_l[mn, pl.ds(j * L, L)] += s
                    return 0
                lax.fori_loop(0, DJ, col, 0)
                return cnt + jnp.where(iota == mn, 16.0, 0.0)

            def general(cnt):
                for r in range(L):
                    br = bv[r]
                    def col2(j, _):
                        sums_l[br, pl.ds(j * L, L)] += \
                            xbuf[g * L + r, pl.ds(j * L, L)]
                        return 0
                    lax.fori_loop(0, DJ, col2, 0)
                    cnt = cnt + jnp.where(iota == br, 1.0, 0.0)
                return cnt

            return lax.cond(mn == mx, uniform, general, cnt_in)
        return lax.fori_loop(0, GRP, grp, cnt_acc)
    cnt_final = lax.fori_loop(0, NCHUNK, p1_chunk, zeros16)
    cnt_l[0, pl.ds(0, L)] = cnt_final

    pltpu.sync_copy(cnt_l.at[0], sh_cslab.at[sid])
    for h in range(2):
        for b8 in range(B // 2):
            pltpu.sync_copy(sums_l.at[h * (B // 2) + b8],
                            sh_slab.at[b8, sid])
        plsc.subcore_barrier()

        @pl.when(sid // (B // 2) == h)
        def _red():
            pltpu.sync_copy(sh_slab.at[sid % (B // 2)], slabv)
            def red(j, _):
                s = zeros16
                for r in range(NS):
                    s = s + slabv[r, pl.ds(j * L, L)]
                redrow[pl.ds(j * L, L)] = s
                return 0
            lax.fori_loop(0, DJ, red, 0)
            pltpu.sync_copy(redrow, sh_sums.at[sid])
        plsc.subcore_barrier()

    pltpu.sync_copy(wt_hbm.at[pl.ds(sid * DL_PER_W, DL_PER_W)], wtv)
    pltpu.sync_copy(sh_sums, sums_v)
    pltpu.sync_copy(sh_cslab, cslabv)
    cnt = zeros16
    for r in range(NS):
        cnt = cnt + cslabv[r, pl.ds(0, L)]
    inv = 1.0 / jnp.maximum(cnt, 1.0)

    def mk_mean(k, _):
        col = plsc.load_gather(sums_v, [iota, jnp.full((L,), k, i32)])
        meanT[k, :] = col * inv
        return 0
    lax.fori_loop(0, D, mk_mean, 0)

    for dl in range(DL_PER_W):
        def mm(k16, acc):
            wv = wtv[dl, pl.ds(k16 * L, L)]
            for j in range(L):
                acc = acc + meanT[k16 * L + j, :] * wv[j]
            return acc
        acc = lax.fori_loop(0, DJ, mm, zeros16)
        trows[0, pl.ds(dl * L, L)] = _tanh(acc)
    pltpu.sync_copy(trows.at[0], sh_slab.at[0, sid])
    plsc.subcore_barrier()

    pltpu.sync_copy(sh_slab.at[0], ttw)
    plsc.subcore_barrier()

    def p2_chunk(c, _):
        pltpu.sync_copy(x_hbm.at[pl.ds(base + c * CHUNK, CHUNK)], xbuf)

        def grp(g, _):
            rows = iota + g * L
            bv = bidx_all[pl.ds(c * CHUNK + g * L, L)]
            mn = lax.reduce_min(bv, (0,))
            mx = lax.reduce_max(bv, (0,))

            def dot16(k16, acc):
                wv16 = jnp.full((L,), k16, i32)
                for j in range(L):
                    dv = jnp.full((L,), k16 * L + j, i32)
                    xv = plsc.load_gather(xbuf, [rows, dv])
                    tv = plsc.load_gather(ttw, [wv16, bv + j * L])
                    acc = acc + xv * tv
                return acc
            dots = lax.fori_loop(0, DJ, dot16, zeros16)
            coef = _sigmoid(dots)

            def uniform():
                def col(j, _):
                    s = zeros16
                    for r in range(L):
                        s = s + coef[r] * xbuf[g * L + r, pl.ds(j * L, L)]
                    out_l[mn, pl.ds(j * L, L)] += s
                    return 0
                lax.fori_loop(0, DJ, col, 0)

            def general():
                for r in range(L):
                    br = bv[r]
                    cr = coef[r]
                    def col2(j, _):
                        out_l[br, pl.ds(j * L, L)] += \
                            cr * xbuf[g * L + r, pl.ds(j * L, L)]
                        return 0
                    lax.fori_loop(0, DJ, col2, 0)

            lax.cond(mn == mx, uniform, general)
            return 0
        lax.fori_loop(0, GRP, grp, 0)
        return 0
    lax.fori_loop(0, NCHUNK, p2_chunk, 0)

    for h in range(2):
        for b8 in range(B // 2):
            pltpu.sync_copy(out_l.at[h * (B // 2) + b8],
                            sh_slab.at[b8, sid])
        plsc.subcore_barrier()

        @pl.when((sid // (B // 2) == h) & (cid == h))
        def _red2():
            pltpu.sync_copy(sh_slab.at[sid % (B // 2)], slabv)
            def red2(j, _):
                s = zeros16
                for r in range(NS):
                    s = s + slabv[r, pl.ds(j * L, L)]
                redrow[pl.ds(j * L, L)] = s
                return 0
            lax.fori_loop(0, DJ, red2, 0)
            pltpu.sync_copy(redrow, out_hbm.at[sid])
        plsc.subcore_barrier()


@jax.jit
def kernel(x, batch, W):
    wt = W.T
    mesh = plsc.VectorSubcoreMesh(core_axis_name="c", subcore_axis_name="s")
    run = pl.kernel(
        _body,
        out_type=jax.ShapeDtypeStruct((B, D), f32),
        mesh=mesh,
        compiler_params=pltpu.CompilerParams(needs_layout_passes=False),
        scratch_types=[
            pltpu.VMEM((CHUNK, D), f32),
            pltpu.VMEM((ROWS_PER_W,), i32),
            pltpu.VMEM((B, D), f32),
            pltpu.VMEM((B, D), f32),
            pltpu.VMEM((1, D), f32),
            pltpu.VMEM((NS, D), f32),
            pltpu.VMEM((D,), f32),
            pltpu.VMEM((NS, D), f32),
            pltpu.VMEM((B, D), f32),
            pltpu.VMEM((D, B), f32),
            pltpu.VMEM((DL_PER_W, D), f32),
            pltpu.VMEM((1, D), f32),
            pltpu.VMEM((NS, D), f32),
            pltpu.VMEM_SHARED((B // 2, NS, D), f32),
            pltpu.VMEM_SHARED((NS, D), f32),
            pltpu.VMEM_SHARED((B, D), f32),
        ],
    )
    return run(x, batch, wt)


# speedup vs baseline: 2.0840x; 2.0840x over previous
import jax
import jax.numpy as jnp
from jax import lax
from jax.experimental import pallas as pl
from jax.experimental.pallas import tpu as pltpu
from jax.experimental.pallas import tpu_sc as plsc

N = 16384
D = 256
B = 16
L = 16
NS = 16
CHUNK = 128
ROWS_PER_W = N // NS
NCHUNK = ROWS_PER_W // CHUNK
GRP = CHUNK // L
DL_PER_W = D // NS
DJ = D // L

f32 = jnp.float32
i32 = jnp.int32


def _sigmoid(v):
    e = jnp.exp(-jnp.abs(v))
    return jnp.where(v >= 0.0, 1.0 / (1.0 + e), e / (1.0 + e))


def _tanh(v):
    e = jnp.exp(-2.0 * jnp.abs(v))
    return jnp.sign(v) * (1.0 - e) / (1.0 + e)


def _body(x_hbm, batch_hbm, wt_hbm, out_hbm,
          xbuf, bidx_all, sums_l, out_l, cnt_l, slabv, redrow, cslabv,
          sums_v, meanT, wtv, trows, ttw, t_rm,
          sh_slab, sh_cslab, sh_sums):
    cid = lax.axis_index("c")
    sid = lax.axis_index("s")
    base = sid * ROWS_PER_W
    iota = lax.iota(i32, L)
    zeros16 = jnp.zeros((L,), f32)

    def zr(r, _):
        def zc(j, _):
            sums_l[r, pl.ds(j * L, L)] = zeros16
            out_l[r, pl.ds(j * L, L)] = zeros16
            return 0
        lax.fori_loop(0, DJ, zc, 0)
        return 0
    lax.fori_loop(0, B, zr, 0)

    pltpu.sync_copy(batch_hbm.at[pl.ds(base, ROWS_PER_W)], bidx_all)

    def p1_chunk(c, cnt_acc):
        pltpu.sync_copy(x_hbm.at[pl.ds(base + c * CHUNK, CHUNK)], xbuf)

        def grp(g, cnt_in):
            bv = bidx_all[pl.ds(c * CHUNK + g * L, L)]
            mn = lax.reduce_min(bv, (0,))
            mx = lax.reduce_max(bv, (0,))

            def uniform(cnt):
                def col(j, _):
                    s = zeros16
                    for r in range(L):
                        s = s + xbuf[g * L + r, pl.ds(j * L, L)]
                    sums_l[mn, pl.ds(j * L, L)] += s
                    return 0
                lax.fori_loop(0, DJ, col, 0)
                return cnt + jnp.where(iota == mn, 16.0, 0.0)

            def general(cnt):
                for r in range(L):
                    br = bv[r]
                    def col2(j, _):
                        sums_l[br, pl.ds(j * L, L)] += \
                            xbuf[g * L + r, pl.ds(j * L, L)]
                        return 0
                    lax.fori_loop(0, DJ, col2, 0)
                    cnt = cnt + jnp.where(iota == br, 1.0, 0.0)
                return cnt

            return lax.cond(mn == mx, uniform, general, cnt_in)
        return lax.fori_loop(0, GRP, grp, cnt_acc)
    cnt_final = lax.fori_loop(0, NCHUNK, p1_chunk, zeros16)
    cnt_l[0, pl.ds(0, L)] = cnt_final

    pltpu.sync_copy(cnt_l.at[0], sh_cslab.at[sid])
    for h in range(2):
        for b8 in range(B // 2):
            pltpu.sync_copy(sums_l.at[h * (B // 2) + b8],
                            sh_slab.at[b8, sid])
        plsc.subcore_barrier()

        @pl.when(sid // (B // 2) == h)
        def _red():
            pltpu.sync_copy(sh_slab.at[sid % (B // 2)], slabv)
            def red(j, _):
                s = zeros16
                for r in range(NS):
                    s = s + slabv[r, pl.ds(j * L, L)]
                redrow[pl.ds(j * L, L)] = s
                return 0
            lax.fori_loop(0, DJ, red, 0)
            pltpu.sync_copy(redrow, sh_sums.at[sid])
        plsc.subcore_barrier()

    pltpu.sync_copy(wt_hbm.at[pl.ds(sid * DL_PER_W, DL_PER_W)], wtv)
    pltpu.sync_copy(sh_sums, sums_v)
    pltpu.sync_copy(sh_cslab, cslabv)
    cnt = zeros16
    for r in range(NS):
        cnt = cnt + cslabv[r, pl.ds(0, L)]
    inv = 1.0 / jnp.maximum(cnt, 1.0)

    def mk_mean(k, _):
        col = plsc.load_gather(sums_v, [iota, jnp.full((L,), k, i32)])
        meanT[k, :] = col * inv
        return 0
    lax.fori_loop(0, D, mk_mean, 0)

    for dl in range(DL_PER_W):
        def mm(k16, acc):
            wv = wtv[dl, pl.ds(k16 * L, L)]
            for j in range(L):
                acc = acc + meanT[k16 * L + j, :] * wv[j]
            return acc
        acc = lax.fori_loop(0, DJ, mm, zeros16)
        plsc.store_scatter(trows, [jnp.zeros((L,), i32), iota * L + dl],
                           _tanh(acc))
    pltpu.sync_copy(trows.at[0], sh_slab.at[0, sid])
    plsc.subcore_barrier()

    pltpu.sync_copy(sh_slab.at[0], ttw)
    plsc.subcore_barrier()
    def t_build(w, _):
        for b in range(B):
            t_rm[b, pl.ds(w * L, L)] = ttw[w, pl.ds(b * L, L)]
        return 0
    lax.fori_loop(0, NS, t_build, 0)

    def p2_chunk(c, _):
        pltpu.sync_copy(x_hbm.at[pl.ds(base + c * CHUNK, CHUNK)], xbuf)

        def grp(g, _):
            bv = bidx_all[pl.ds(c * CHUNK + g * L, L)]
            mn = lax.reduce_min(bv, (0,))
            mx = lax.reduce_max(bv, (0,))

            zt = (zeros16,) * L

            def uniform_dots():
                def jcl(jc, ps):
                    tv = t_rm[mn, pl.ds(jc * L, L)]
                    return tuple(
                        ps[r] + xbuf[g * L + r, pl.ds(jc * L, L)] * tv
                        for r in range(L))
                return lax.fori_loop(0, DJ, jcl, zt)

            def general_dots():
                def jcl(jc, ps):
                    out = []
                    for r in range(L):
                        tvr = t_rm[bv[r], pl.ds(jc * L, L)]
                        out.append(
                            ps[r] + xbuf[g * L + r, pl.ds(jc * L, L)] * tvr)
                    return tuple(out)
                return lax.fori_loop(0, DJ, jcl, zt)

            psums = lax.cond(mn == mx, uniform_dots, general_dots)
            dots = zeros16
            for r in range(L):
                dr = lax.reduce_sum(psums[r], (0,))
                dots = jnp.where(iota == r, dr, dots)
            coef = _sigmoid(dots)

            def uniform():
                def col(j, _):
                    s = zeros16
                    for r in range(L):
                        s = s + coef[r] * xbuf[g * L + r, pl.ds(j * L, L)]
                    out_l[mn, pl.ds(j * L, L)] += s
                    return 0
                lax.fori_loop(0, DJ, col, 0)

            def general():
                for r in range(L):
                    br = bv[r]
                    cr = coef[r]
                    def col2(j, _):
                        out_l[br, pl.ds(j * L, L)] += \
                            cr * xbuf[g * L + r, pl.ds(j * L, L)]
                        return 0
                    lax.fori_loop(0, DJ, col2, 0)

            lax.cond(mn == mx, uniform, general)
            return 0
        lax.fori_loop(0, GRP, grp, 0)
        return 0
    lax.fori_loop(0, NCHUNK, p2_chunk, 0)

    for h in range(2):
        for b8 in range(B // 2):
            pltpu.sync_copy(out_l.at[h * (B // 2) + b8],
                            sh_slab.at[b8, sid])
        plsc.subcore_barrier()

        @pl.when((sid // (B // 2) == h) & (cid == h))
        def _red2():
            pltpu.sync_copy(sh_slab.at[sid % (B // 2)], slabv)
            def red2(j, _):
                s = zeros16
                for r in range(NS):
                    s = s + slabv[r, pl.ds(j * L, L)]
                redrow[pl.ds(j * L, L)] = s
                return 0
            lax.fori_loop(0, DJ, red2, 0)
            pltpu.sync_copy(redrow, out_hbm.at[sid])
        plsc.subcore_barrier()


@jax.jit
def kernel(x, batch, W):
    wt = W.T
    mesh = plsc.VectorSubcoreMesh(core_axis_name="c", subcore_axis_name="s")
    run = pl.kernel(
        _body,
        out_type=jax.ShapeDtypeStruct((B, D), f32),
        mesh=mesh,
        compiler_params=pltpu.CompilerParams(needs_layout_passes=False),
        scratch_types=[
            pltpu.VMEM((CHUNK, D), f32),
            pltpu.VMEM((ROWS_PER_W,), i32),
            pltpu.VMEM((B, D), f32),
            pltpu.VMEM((B, D), f32),
            pltpu.VMEM((1, D), f32),
            pltpu.VMEM((NS, D), f32),
            pltpu.VMEM((D,), f32),
            pltpu.VMEM((NS, D), f32),
            pltpu.VMEM((B, D), f32),
            pltpu.VMEM((D, B), f32),
            pltpu.VMEM((DL_PER_W, D), f32),
            pltpu.VMEM((1, D), f32),
            pltpu.VMEM((NS, D), f32),
            pltpu.VMEM((B, D), f32),
            pltpu.VMEM_SHARED((B // 2, NS, D), f32),
            pltpu.VMEM_SHARED((NS, D), f32),
            pltpu.VMEM_SHARED((B, D), f32),
        ],
    )
    return run(x, batch, wt)


# speedup vs baseline: 2.7544x; 1.3217x over previous
import jax
import jax.numpy as jnp
from jax import lax
from jax.experimental import pallas as pl
from jax.experimental.pallas import tpu as pltpu
from jax.experimental.pallas import tpu_sc as plsc

N = 16384
D = 256
B = 16
L = 16
NS = 16
NC = 2
NW = NC * NS
CHUNK = 128
ROWS_PER_W = N // NW
NCHUNK = ROWS_PER_W // CHUNK
GRP = CHUNK // L
DL_PER_W = D // NS
DJ = D // L

f32 = jnp.float32
i32 = jnp.int32


def _sigmoid(v):
    e = jnp.exp(-jnp.abs(v))
    return jnp.where(v >= 0.0, 1.0 / (1.0 + e), e / (1.0 + e))


def _tanh(v):
    e = jnp.exp(-2.0 * jnp.abs(v))
    return jnp.sign(v) * (1.0 - e) / (1.0 + e)


def _zero_acc(acc):
    zeros16 = jnp.zeros((L,), f32)
    def zr(r, _):
        def zc(j, _):
            acc[r, pl.ds(j * L, L)] = zeros16
            return 0
        lax.fori_loop(0, DJ, zc, 0)
        return 0
    lax.fori_loop(0, B, zr, 0)


def _accum_rows(xbuf, bidx_all, acc, c, weights=None):
    raise NotImplementedError


def _bodyA(x_hbm, batch_hbm, pa_hbm,
           xbuf, bidx_all, sums_l, cnt_l, slabv, redrow, cslabv,
           sh_slab, sh_cslab):
    cid = lax.axis_index("c")
    sid = lax.axis_index("s")
    base = (cid * NS + sid) * ROWS_PER_W
    iota = lax.iota(i32, L)
    zeros16 = jnp.zeros((L,), f32)

    _zero_acc(sums_l)
    pltpu.sync_copy(batch_hbm.at[pl.ds(base, ROWS_PER_W)], bidx_all)

    def p1_chunk(c, cnt_acc):
        pltpu.sync_copy(x_hbm.at[pl.ds(base + c * CHUNK, CHUNK)], xbuf)

        def grp(g, cnt_in):
            bv = bidx_all[pl.ds(c * CHUNK + g * L, L)]
            mn = lax.reduce_min(bv, (0,))
            mx = lax.reduce_max(bv, (0,))

            def uniform(cnt):
                def col(j, _):
                    s = zeros16
                    for r in range(L):
                        s = s + xbuf[g * L + r, pl.ds(j * L, L)]
                    sums_l[mn, pl.ds(j * L, L)] += s
                    return 0
                lax.fori_loop(0, DJ, col, 0)
                return cnt + jnp.where(iota == mn, 16.0, 0.0)

            def general(cnt):
                for r in range(L):
                    br = bv[r]
                    def col2(j, _):
                        sums_l[br, pl.ds(j * L, L)] += \
                            xbuf[g * L + r, pl.ds(j * L, L)]
                        return 0
                    lax.fori_loop(0, DJ, col2, 0)
                    cnt = cnt + jnp.where(iota == br, 1.0, 0.0)
                return cnt

            return lax.cond(mn == mx, uniform, general, cnt_in)
        return lax.fori_loop(0, GRP, grp, cnt_acc)
    cnt_final = lax.fori_loop(0, NCHUNK, p1_chunk, zeros16)
    cnt_l[0, pl.ds(0, L)] = cnt_final

    pltpu.sync_copy(cnt_l.at[0], sh_cslab.at[sid])
    for h in range(2):
        for b8 in range(B // 2):
            pltpu.sync_copy(sums_l.at[h * (B // 2) + b8], sh_slab.at[b8, sid])
        plsc.subcore_barrier()

        @pl.when(sid // (B // 2) == h)
        def _red():
            pltpu.sync_copy(sh_slab.at[sid % (B // 2)], slabv)
            def red(j, _):
                s = zeros16
                for r in range(NS):
                    s = s + slabv[r, pl.ds(j * L, L)]
                redrow[pl.ds(j * L, L)] = s
                return 0
            lax.fori_loop(0, DJ, red, 0)
            pltpu.sync_copy(redrow, pa_hbm.at[cid * (B + 1) + sid])
        plsc.subcore_barrier()

    @pl.when(sid == 0)
    def _cw():
        pltpu.sync_copy(sh_cslab, cslabv)
        cnt = zeros16
        for r in range(NS):
            cnt = cnt + cslabv[r, pl.ds(0, L)]
        redrow[pl.ds(0, L)] = cnt
        pltpu.sync_copy(redrow, pa_hbm.at[cid * (B + 1) + B])


def _bodyB(x_hbm, batch_hbm, wt_hbm, pa_hbm, po_hbm,
           xbuf, bidx_all, out_l, pv, slabv, redrow,
           sums_v, meanT, wtv, trows, ttw, t_rm,
           sh_slab):
    cid = lax.axis_index("c")
    sid = lax.axis_index("s")
    base = (cid * NS + sid) * ROWS_PER_W
    iota = lax.iota(i32, L)
    zeros16 = jnp.zeros((L,), f32)

    _zero_acc(out_l)
    pltpu.sync_copy(batch_hbm.at[pl.ds(base, ROWS_PER_W)], bidx_all)
    pltpu.sync_copy(wt_hbm.at[pl.ds(sid * DL_PER_W, DL_PER_W)], wtv)
    pltpu.sync_copy(pa_hbm, pv)

    def addp(b, _):
        def jc(j, _):
            sums_v[b, pl.ds(j * L, L)] = (
                pv[b, pl.ds(j * L, L)] + pv[B + 1 + b, pl.ds(j * L, L)])
            return 0
        lax.fori_loop(0, DJ, jc, 0)
        return 0
    lax.fori_loop(0, B, addp, 0)
    cnt = pv[B, pl.ds(0, L)] + pv[2 * B + 1, pl.ds(0, L)]
    inv = 1.0 / jnp.maximum(cnt, 1.0)

    def mk_mean(k, _):
        col = plsc.load_gather(sums_v, [iota, jnp.full((L,), k, i32)])
        meanT[k, :] = col * inv
        return 0
    lax.fori_loop(0, D, mk_mean, 0)

    for dl in range(DL_PER_W):
        def mm(k16, acc):
            wv = wtv[dl, pl.ds(k16 * L, L)]
            for j in range(L):
                acc = acc + meanT[k16 * L + j, :] * wv[j]
            return acc
        acc = lax.fori_loop(0, DJ, mm, zeros16)
        plsc.store_scatter(trows, [jnp.zeros((L,), i32), iota * L + dl],
                           _tanh(acc))
    pltpu.sync_copy(trows.at[0], sh_slab.at[0, sid])
    plsc.subcore_barrier()
    pltpu.sync_copy(sh_slab.at[0], ttw)
    plsc.subcore_barrier()
    def t_build(w, _):
        for b in range(B):
            t_rm[b, pl.ds(w * L, L)] = ttw[w, pl.ds(b * L, L)]
        return 0
    lax.fori_loop(0, NS, t_build, 0)

    def p2_chunk(c, _):
        pltpu.sync_copy(x_hbm.at[pl.ds(base + c * CHUNK, CHUNK)], xbuf)

        def grp(g, _):
            bv = bidx_all[pl.ds(c * CHUNK + g * L, L)]
            mn = lax.reduce_min(bv, (0,))
            mx = lax.reduce_max(bv, (0,))
            zt = (zeros16,) * L

            def uniform_dots():
                def jcl(jc, ps):
                    tv = t_rm[mn, pl.ds(jc * L, L)]
                    return tuple(
                        ps[r] + xbuf[g * L + r, pl.ds(jc * L, L)] * tv
                        for r in range(L))
                return lax.fori_loop(0, DJ, jcl, zt)

            def general_dots():
                def jcl(jc, ps):
                    out = []
                    for r in range(L):
                        tvr = t_rm[bv[r], pl.ds(jc * L, L)]
                        out.append(
                            ps[r] + xbuf[g * L + r, pl.ds(jc * L, L)] * tvr)
                    return tuple(out)
                return lax.fori_loop(0, DJ, jcl, zt)

            psums = lax.cond(mn == mx, uniform_dots, general_dots)
            dots = zeros16
            for r in range(L):
                dr = lax.reduce_sum(psums[r], (0,))
                dots = jnp.where(iota == r, dr, dots)
            coef = _sigmoid(dots)

            def uniform():
                def col(j, _):
                    s = zeros16
                    for r in range(L):
                        s = s + coef[r] * xbuf[g * L + r, pl.ds(j * L, L)]
                    out_l[mn, pl.ds(j * L, L)] += s
                    return 0
                lax.fori_loop(0, DJ, col, 0)

            def general():
                for r in range(L):
                    br = bv[r]
                    cr = coef[r]
                    def col2(j, _):
                        out_l[br, pl.ds(j * L, L)] += \
                            cr * xbuf[g * L + r, pl.ds(j * L, L)]
                        return 0
                    lax.fori_loop(0, DJ, col2, 0)

            lax.cond(mn == mx, uniform, general)
            return 0
        lax.fori_loop(0, GRP, grp, 0)
        return 0
    lax.fori_loop(0, NCHUNK, p2_chunk, 0)

    for h in range(2):
        for b8 in range(B // 2):
            pltpu.sync_copy(out_l.at[h * (B // 2) + b8], sh_slab.at[b8, sid])
        plsc.subcore_barrier()

        @pl.when(sid // (B // 2) == h)
        def _red2():
            pltpu.sync_copy(sh_slab.at[sid % (B // 2)], slabv)
            def red2(j, _):
                s = zeros16
                for r in range(NS):
                    s = s + slabv[r, pl.ds(j * L, L)]
                redrow[pl.ds(j * L, L)] = s
                return 0
            lax.fori_loop(0, DJ, red2, 0)
            pltpu.sync_copy(redrow, po_hbm.at[cid * B + sid])
        plsc.subcore_barrier()


def _bodyC(po_hbm, out_hbm, cbuf):
    cid = lax.axis_index("c")
    sid = lax.axis_index("s")
    zeros16 = jnp.zeros((L,), f32)

    @pl.when(cid == 0)
    def _c():
        pltpu.sync_copy(po_hbm.at[sid], cbuf.at[0])
        pltpu.sync_copy(po_hbm.at[B + sid], cbuf.at[1])
        def jc(j, _):
            cbuf[0, pl.ds(j * L, L)] += cbuf[1, pl.ds(j * L, L)]
            return 0
        lax.fori_loop(0, DJ, jc, 0)
        pltpu.sync_copy(cbuf.at[0], out_hbm.at[sid])


@jax.jit
def kernel(x, batch, W):
    wt = W.T
    mesh = plsc.VectorSubcoreMesh(core_axis_name="c", subcore_axis_name="s")
    cp = pltpu.CompilerParams(needs_layout_passes=False)

    runA = pl.kernel(
        _bodyA,
        out_type=jax.ShapeDtypeStruct((NC * (B + 1), D), f32),
        mesh=mesh, compiler_params=cp,
        scratch_types=[
            pltpu.VMEM((CHUNK, D), f32),
            pltpu.VMEM((ROWS_PER_W,), i32),
            pltpu.VMEM((B, D), f32),
            pltpu.VMEM((1, D), f32),
            pltpu.VMEM((NS, D), f32),
            pltpu.VMEM((D,), f32),
            pltpu.VMEM((NS, D), f32),
            pltpu.VMEM_SHARED((B // 2, NS, D), f32),
            pltpu.VMEM_SHARED((NS, D), f32),
        ],
    )
    pa = runA(x, batch)

    runB = pl.kernel(
        _bodyB,
        out_type=jax.ShapeDtypeStruct((NC * B, D), f32),
        mesh=mesh, compiler_params=cp,
        scratch_types=[
            pltpu.VMEM((CHUNK, D), f32),
            pltpu.VMEM((ROWS_PER_W,), i32),
            pltpu.VMEM((B, D), f32),
            pltpu.VMEM((NC * (B + 1), D), f32),
            pltpu.VMEM((NS, D), f32),
            pltpu.VMEM((D,), f32),
            pltpu.VMEM((B, D), f32),
            pltpu.VMEM((D, B), f32),
            pltpu.VMEM((DL_PER_W, D), f32),
            pltpu.VMEM((1, D), f32),
            pltpu.VMEM((NS, D), f32),
            pltpu.VMEM((B, D), f32),
            pltpu.VMEM_SHARED((B // 2, NS, D), f32),
        ],
    )
    po = runB(x, batch, wt, pa)

    runC = pl.kernel(
        _bodyC,
        out_type=jax.ShapeDtypeStruct((B, D), f32),
        mesh=mesh, compiler_params=cp,
        scratch_types=[
            pltpu.VMEM((2, D), f32),
        ],
    )
    return runC(po)


# speedup vs baseline: 2.9771x; 1.0809x over previous
import jax
import jax.numpy as jnp
from jax import lax
from jax.experimental import pallas as pl
from jax.experimental.pallas import tpu as pltpu
from jax.experimental.pallas import tpu_sc as plsc

N = 16384
D = 256
B = 16
L = 16
NS = 16
NC = 2
NW = NC * NS
CHUNK = 64
ROWS_PER_W = N // NW
NCHUNK = ROWS_PER_W // CHUNK
GRP = CHUNK // L
DL_PER_W = D // NS
DJ = D // L

f32 = jnp.float32
i32 = jnp.int32


def _sigmoid(v):
    e = jnp.exp(-jnp.abs(v))
    return jnp.where(v >= 0.0, 1.0 / (1.0 + e), e / (1.0 + e))


def _tanh(v):
    e = jnp.exp(-2.0 * jnp.abs(v))
    return jnp.sign(v) * (1.0 - e) / (1.0 + e)


def _zero_acc(acc):
    zeros16 = jnp.zeros((L,), f32)
    def zr(r, _):
        def zc(j, _):
            acc[r, pl.ds(j * L, L)] = zeros16
            return 0
        lax.fori_loop(0, DJ, zc, 0)
        return 0
    lax.fori_loop(0, B, zr, 0)


def _accum_rows(xbuf, bidx_all, acc, c, weights=None):
    raise NotImplementedError


def _bodyA(x_hbm, batch_hbm, pa_hbm,
           xb0, xb1, bidx_all, sums_l, cnt_l, slabv, redrow, cslabv,
           sem0, sem1,
           sh_slab, sh_cslab):
    cid = lax.axis_index("c")
    sid = lax.axis_index("s")
    base = (cid * NS + sid) * ROWS_PER_W
    iota = lax.iota(i32, L)
    zeros16 = jnp.zeros((L,), f32)
    bufs = [xb0, xb1]
    sems = [sem0, sem1]

    _zero_acc(sums_l)
    pltpu.sync_copy(batch_hbm.at[pl.ds(base, ROWS_PER_W)], bidx_all)

    def p1_chunk(c, xbuf, cnt_acc):
        def grp(g, cnt_in):
            bv = bidx_all[pl.ds(c * CHUNK + g * L, L)]
            mn = lax.reduce_min(bv, (0,))
            mx = lax.reduce_max(bv, (0,))

            def uniform(cnt):
                def col(j, _):
                    s = zeros16
                    for r in range(L):
                        s = s + xbuf[g * L + r, pl.ds(j * L, L)]
                    sums_l[mn, pl.ds(j * L, L)] += s
                    return 0
                lax.fori_loop(0, DJ, col, 0)
                return cnt + jnp.where(iota == mn, 16.0, 0.0)

            def general(cnt):
                for r in range(L):
                    br = bv[r]
                    def col2(j, _):
                        sums_l[br, pl.ds(j * L, L)] += \
                            xbuf[g * L + r, pl.ds(j * L, L)]
                        return 0
                    lax.fori_loop(0, DJ, col2, 0)
                    cnt = cnt + jnp.where(iota == br, 1.0, 0.0)
                return cnt

            return lax.cond(mn == mx, uniform, general, cnt_in)
        return lax.fori_loop(0, GRP, grp, cnt_acc)

    pltpu.async_copy(x_hbm.at[pl.ds(base, CHUNK)], bufs[0], sems[0])
    pltpu.async_copy(x_hbm.at[pl.ds(base + CHUNK, CHUNK)], bufs[1], sems[1])

    def pair(c2, cnt_acc):
        for b in range(2):
            c = 2 * c2 + b
            pltpu.make_async_copy(
                x_hbm.at[pl.ds(base, CHUNK)], bufs[b], sems[b]).wait()
            cnt_acc = p1_chunk(c, bufs[b], cnt_acc)

            @pl.when(c + 2 < NCHUNK)
            def _pf():
                pltpu.async_copy(
                    x_hbm.at[pl.ds(base + (c + 2) * CHUNK, CHUNK)],
                    bufs[b], sems[b])
        return cnt_acc
    cnt_final = lax.fori_loop(0, NCHUNK // 2, pair, zeros16)
    cnt_l[0, pl.ds(0, L)] = cnt_final

    pltpu.sync_copy(cnt_l.at[0], sh_cslab.at[sid])
    for h in range(2):
        for b8 in range(B // 2):
            pltpu.sync_copy(sums_l.at[h * (B // 2) + b8], sh_slab.at[b8, sid])
        plsc.subcore_barrier()

        @pl.when(sid // (B // 2) == h)
        def _red():
            pltpu.sync_copy(sh_slab.at[sid % (B // 2)], slabv)
            def red(j, _):
                s = zeros16
                for r in range(NS):
                    s = s + slabv[r, pl.ds(j * L, L)]
                redrow[pl.ds(j * L, L)] = s
                return 0
            lax.fori_loop(0, DJ, red, 0)
            pltpu.sync_copy(redrow, pa_hbm.at[cid * 2 * B + sid])
        plsc.subcore_barrier()

    @pl.when(sid == 0)
    def _cw():
        pltpu.sync_copy(sh_cslab, cslabv)
        cnt = zeros16
        for r in range(NS):
            cnt = cnt + cslabv[r, pl.ds(0, L)]
        redrow[pl.ds(0, L)] = cnt
        pltpu.sync_copy(redrow, pa_hbm.at[cid * 2 * B + B])


def _bodyB(x_hbm, batch_hbm, wt_hbm, pa_hbm, po_hbm,
           xb0, xb1, bidx_all, out_l, slabv, redrow,
           sums_v, meanT, wtv, trows, ttw, t_rm, sem0, sem1,
           sh_slab):
    cid = lax.axis_index("c")
    sid = lax.axis_index("s")
    base = (cid * NS + sid) * ROWS_PER_W
    iota = lax.iota(i32, L)
    zeros16 = jnp.zeros((L,), f32)

    bufs = [xb0, xb1]
    sems = [sem0, sem1]
    descs = [None, None]
    descs[0] = pltpu.async_copy(
        x_hbm.at[pl.ds(base, CHUNK)], bufs[0], sems[0])

    _zero_acc(out_l)
    pltpu.sync_copy(batch_hbm.at[pl.ds(base, ROWS_PER_W)], bidx_all)
    pltpu.sync_copy(wt_hbm.at[pl.ds(sid * DL_PER_W, DL_PER_W)], wtv)
    pltpu.sync_copy(pa_hbm.at[pl.ds(0, B)], sums_v)
    pltpu.sync_copy(pa_hbm.at[pl.ds(2 * B, B)], slabv)
    pltpu.sync_copy(pa_hbm.at[B], ttw.at[0])
    pltpu.sync_copy(pa_hbm.at[3 * B], ttw.at[1])

    def addp(b, _):
        def jc(j, _):
            sums_v[b, pl.ds(j * L, L)] += slabv[b, pl.ds(j * L, L)]
            return 0
        lax.fori_loop(0, DJ, jc, 0)
        return 0
    lax.fori_loop(0, B, addp, 0)
    cnt = ttw[0, pl.ds(0, L)] + ttw[1, pl.ds(0, L)]
    inv = 1.0 / jnp.maximum(cnt, 1.0)

    def mk_mean(k, _):
        col = plsc.load_gather(sums_v, [iota, jnp.full((L,), k, i32)])
        meanT[k, :] = col * inv
        return 0
    lax.fori_loop(0, D, mk_mean, 0)

    for dl in range(DL_PER_W):
        def mm(k16, acc):
            wv = wtv[dl, pl.ds(k16 * L, L)]
            for j in range(L):
                acc = acc + meanT[k16 * L + j, :] * wv[j]
            return acc
        acc = lax.fori_loop(0, DJ, mm, zeros16)
        plsc.store_scatter(trows, [jnp.zeros((L,), i32), iota * L + dl],
                           _tanh(acc))
    pltpu.sync_copy(trows.at[0], sh_slab.at[0, sid])
    plsc.subcore_barrier()
    pltpu.sync_copy(sh_slab.at[0], ttw)
    plsc.subcore_barrier()
    def t_build(w, _):
        for b in range(B):
            t_rm[b, pl.ds(w * L, L)] = ttw[w, pl.ds(b * L, L)]
        return 0
    lax.fori_loop(0, NS, t_build, 0)

    def p2_chunk(c, xbuf):
        def grp(g, _):
            bv = bidx_all[pl.ds(c * CHUNK + g * L, L)]
            mn = lax.reduce_min(bv, (0,))
            mx = lax.reduce_max(bv, (0,))
            zt = (zeros16,) * L

            def uniform_dots():
                def jcl(jc, ps):
                    tv = t_rm[mn, pl.ds(jc * L, L)]
                    return tuple(
                        ps[r] + xbuf[g * L + r, pl.ds(jc * L, L)] * tv
                        for r in range(L))
                return lax.fori_loop(0, DJ, jcl, zt)

            def general_dots():
                def jcl(jc, ps):
                    out = []
                    for r in range(L):
                        tvr = t_rm[bv[r], pl.ds(jc * L, L)]
                        out.append(
                            ps[r] + xbuf[g * L + r, pl.ds(jc * L, L)] * tvr)
                    return tuple(out)
                return lax.fori_loop(0, DJ, jcl, zt)

            psums = lax.cond(mn == mx, uniform_dots, general_dots)
            dots = zeros16
            for r in range(L):
                dr = lax.reduce_sum(psums[r], (0,))
                dots = jnp.where(iota == r, dr, dots)
            coef = _sigmoid(dots)

            def uniform():
                def col(j, _):
                    s = zeros16
                    for r in range(L):
                        s = s + coef[r] * xbuf[g * L + r, pl.ds(j * L, L)]
                    out_l[mn, pl.ds(j * L, L)] += s
                    return 0
                lax.fori_loop(0, DJ, col, 0)

            def general():
                for r in range(L):
                    br = bv[r]
                    cr = coef[r]
                    def col2(j, _):
                        out_l[br, pl.ds(j * L, L)] += \
                            cr * xbuf[g * L + r, pl.ds(j * L, L)]
                        return 0
                    lax.fori_loop(0, DJ, col2, 0)

            lax.cond(mn == mx, uniform, general)
            return 0
        lax.fori_loop(0, GRP, grp, 0)

    pltpu.async_copy(x_hbm.at[pl.ds(base + CHUNK, CHUNK)], bufs[1], sems[1])

    def pair2(c2, _):
        for b in range(2):
            c = 2 * c2 + b
            pltpu.make_async_copy(
                x_hbm.at[pl.ds(base, CHUNK)], bufs[b], sems[b]).wait()
            p2_chunk(c, bufs[b])

            @pl.when(c + 2 < NCHUNK)
            def _pf():
                pltpu.async_copy(
                    x_hbm.at[pl.ds(base + (c + 2) * CHUNK, CHUNK)],
                    bufs[b], sems[b])
        return 0
    lax.fori_loop(0, NCHUNK // 2, pair2, 0)

    for h in range(2):
        for b8 in range(B // 2):
            pltpu.sync_copy(out_l.at[h * (B // 2) + b8], sh_slab.at[b8, sid])
        plsc.subcore_barrier()

        @pl.when(sid // (B // 2) == h)
        def _red2():
            pltpu.sync_copy(sh_slab.at[sid % (B // 2)], slabv)
            def red2(j, _):
                s = zeros16
                for r in range(NS):
                    s = s + slabv[r, pl.ds(j * L, L)]
                redrow[pl.ds(j * L, L)] = s
                return 0
            lax.fori_loop(0, DJ, red2, 0)
            pltpu.sync_copy(redrow, po_hbm.at[cid * B + sid])
        plsc.subcore_barrier()


def _bodyC(po_hbm, out_hbm, cbuf):
    cid = lax.axis_index("c")
    sid = lax.axis_index("s")
    zeros16 = jnp.zeros((L,), f32)

    @pl.when(cid == 0)
    def _c():
        pltpu.sync_copy(po_hbm.at[sid], cbuf.at[0])
        pltpu.sync_copy(po_hbm.at[B + sid], cbuf.at[1])
        def jc(j, _):
            cbuf[0, pl.ds(j * L, L)] += cbuf[1, pl.ds(j * L, L)]
            return 0
        lax.fori_loop(0, DJ, jc, 0)
        pltpu.sync_copy(cbuf.at[0], out_hbm.at[sid])


@jax.jit
def kernel(x, batch, W):
    wt = W.T
    mesh = plsc.VectorSubcoreMesh(core_axis_name="c", subcore_axis_name="s")
    cp = pltpu.CompilerParams(needs_layout_passes=False)

    runA = pl.kernel(
        _bodyA,
        out_type=jax.ShapeDtypeStruct((NC * 2 * B, D), f32),
        mesh=mesh, compiler_params=cp,
        scratch_types=[
            pltpu.VMEM((CHUNK, D), f32),
            pltpu.VMEM((CHUNK, D), f32),
            pltpu.VMEM((ROWS_PER_W,), i32),
            pltpu.VMEM((B, D), f32),
            pltpu.VMEM((1, D), f32),
            pltpu.VMEM((NS, D), f32),
            pltpu.VMEM((D,), f32),
            pltpu.VMEM((NS, D), f32),
            pltpu.SemaphoreType.DMA,
            pltpu.SemaphoreType.DMA,
            pltpu.VMEM_SHARED((B // 2, NS, D), f32),
            pltpu.VMEM_SHARED((NS, D), f32),
        ],
    )
    pa = runA(x, batch)

    runB = pl.kernel(
        _bodyB,
        out_type=jax.ShapeDtypeStruct((NC * B, D), f32),
        mesh=mesh, compiler_params=cp,
        scratch_types=[
            pltpu.VMEM((CHUNK, D), f32),
            pltpu.VMEM((CHUNK, D), f32),
            pltpu.VMEM((ROWS_PER_W,), i32),
            pltpu.VMEM((B, D), f32),
            pltpu.VMEM((NS, D), f32),
            pltpu.VMEM((D,), f32),
            pltpu.VMEM((B, D), f32),
            pltpu.VMEM((D, B), f32),
            pltpu.VMEM((DL_PER_W, D), f32),
            pltpu.VMEM((1, D), f32),
            pltpu.VMEM((NS, D), f32),
            pltpu.VMEM((B, D), f32),
            pltpu.SemaphoreType.DMA,
            pltpu.SemaphoreType.DMA,
            pltpu.VMEM_SHARED((B // 2, NS, D), f32),
        ],
    )
    po = runB(x, batch, wt, pa)

    runC = pl.kernel(
        _bodyC,
        out_type=jax.ShapeDtypeStruct((B, D), f32),
        mesh=mesh, compiler_params=cp,
        scratch_types=[
            pltpu.VMEM((2, D), f32),
        ],
    )
    return runC(po)


# speedup vs baseline: 2.9841x; 1.0023x over previous
import jax
import jax.numpy as jnp
from jax import lax
from jax.experimental import pallas as pl
from jax.experimental.pallas import tpu as pltpu
from jax.experimental.pallas import tpu_sc as plsc

N = 16384
D = 256
B = 16
L = 16
NS = 16
NC = 2
NW = NC * NS
CHUNK = 64
ROWS_P1 = N // NS
NCHUNK1 = ROWS_P1 // CHUNK
ROWS_P2 = N // NW
NCHUNK2 = ROWS_P2 // CHUNK
GRP = CHUNK // L
DL_PER_W = D // NS
DJ = D // L

f32 = jnp.float32
i32 = jnp.int32


def _sigmoid(v):
    e = jnp.exp(-jnp.abs(v))
    return jnp.where(v >= 0.0, 1.0 / (1.0 + e), e / (1.0 + e))


def _tanh(v):
    e = jnp.exp(-2.0 * jnp.abs(v))
    return jnp.sign(v) * (1.0 - e) / (1.0 + e)


def _zero_acc(acc):
    zeros16 = jnp.zeros((L,), f32)
    def zr(r, _):
        def zc(j, _):
            acc[r, pl.ds(j * L, L)] = zeros16
            return 0
        lax.fori_loop(0, DJ, zc, 0)
        return 0
    lax.fori_loop(0, B, zr, 0)


def _bodyAB(x_hbm, batch_hbm, wt_hbm, po_hbm,
            xb0, xb1, bidx1, bidx2, sums_l, out_l, cnt_l, slabv, redrow,
            cslabv, sums_v, meanT, wtv, trows, ttw, t_rm, sem0, sem1,
            sh_slab, sh_cslab, sh_sums):
    cid = lax.axis_index("c")
    sid = lax.axis_index("s")
    base1 = sid * ROWS_P1
    base2 = (cid * NS + sid) * ROWS_P2
    iota = lax.iota(i32, L)
    zeros16 = jnp.zeros((L,), f32)
    bufs = [xb0, xb1]
    sems = [sem0, sem1]

    pltpu.async_copy(x_hbm.at[pl.ds(base1, CHUNK)], bufs[0], sems[0])
    pltpu.async_copy(x_hbm.at[pl.ds(base1 + CHUNK, CHUNK)], bufs[1], sems[1])

    _zero_acc(sums_l)
    _zero_acc(out_l)
    pltpu.sync_copy(batch_hbm.at[pl.ds(base1, ROWS_P1)], bidx1)
    pltpu.sync_copy(batch_hbm.at[pl.ds(base2, ROWS_P2)], bidx2)
    pltpu.sync_copy(wt_hbm.at[pl.ds(sid * DL_PER_W, DL_PER_W)], wtv)

    def p1_chunk(c, xbuf, cnt_acc):
        def grp(g, cnt_in):
            bv = bidx1[pl.ds(c * CHUNK + g * L, L)]
            mn = lax.reduce_min(bv, (0,))
            mx = lax.reduce_max(bv, (0,))

            def uniform(cnt):
                def col(j, _):
                    s = zeros16
                    for r in range(L):
                        s = s + xbuf[g * L + r, pl.ds(j * L, L)]
                    sums_l[mn, pl.ds(j * L, L)] += s
                    return 0
                lax.fori_loop(0, DJ, col, 0)
                return cnt + jnp.where(iota == mn, 16.0, 0.0)

            def general(cnt):
                for r in range(L):
                    br = bv[r]
                    def col2(j, _):
                        sums_l[br, pl.ds(j * L, L)] += \
                            xbuf[g * L + r, pl.ds(j * L, L)]
                        return 0
                    lax.fori_loop(0, DJ, col2, 0)
                    cnt = cnt + jnp.where(iota == br, 1.0, 0.0)
                return cnt

            return lax.cond(mn == mx, uniform, general, cnt_in)
        return lax.fori_loop(0, GRP, grp, cnt_acc)

    def pair1(c2, cnt_acc):
        for b in range(2):
            c = 2 * c2 + b
            pltpu.make_async_copy(
                x_hbm.at[pl.ds(base1, CHUNK)], bufs[b], sems[b]).wait()
            cnt_acc = p1_chunk(c, bufs[b], cnt_acc)

            @pl.when(c + 2 < NCHUNK1)
            def _pf():
                pltpu.async_copy(
                    x_hbm.at[pl.ds(base1 + (c + 2) * CHUNK, CHUNK)],
                    bufs[b], sems[b])
        return cnt_acc
    cnt_final = lax.fori_loop(0, NCHUNK1 // 2, pair1, zeros16)
    cnt_l[0, pl.ds(0, L)] = cnt_final

    pltpu.sync_copy(cnt_l.at[0], sh_cslab.at[sid])
    for h in range(2):
        for b8 in range(B // 2):
            pltpu.sync_copy(sums_l.at[h * (B // 2) + b8], sh_slab.at[b8, sid])
        plsc.subcore_barrier()

        @pl.when(sid // (B // 2) == h)
        def _red():
            pltpu.sync_copy(sh_slab.at[sid % (B // 2)], slabv)
            def red(j, _):
                s = zeros16
                for r in range(NS):
                    s = s + slabv[r, pl.ds(j * L, L)]
                redrow[pl.ds(j * L, L)] = s
                return 0
            lax.fori_loop(0, DJ, red, 0)
            pltpu.sync_copy(redrow, sh_sums.at[sid])
        plsc.subcore_barrier()

    pltpu.async_copy(x_hbm.at[pl.ds(base2, CHUNK)], bufs[0], sems[0])
    pltpu.async_copy(x_hbm.at[pl.ds(base2 + CHUNK, CHUNK)], bufs[1], sems[1])

    pltpu.sync_copy(sh_sums, sums_v)
    pltpu.sync_copy(sh_cslab, cslabv)
    cnt = zeros16
    for r in range(NS):
        cnt = cnt + cslabv[r, pl.ds(0, L)]
    inv = 1.0 / jnp.maximum(cnt, 1.0)

    def mk_mean(k, _):
        col = plsc.load_gather(sums_v, [iota, jnp.full((L,), k, i32)])
        meanT[k, :] = col * inv
        return 0
    lax.fori_loop(0, D, mk_mean, 0)

    for dl in range(DL_PER_W):
        def mm(k16, acc):
            wv = wtv[dl, pl.ds(k16 * L, L)]
            for j in range(L):
                acc = acc + meanT[k16 * L + j, :] * wv[j]
            return acc
        acc = lax.fori_loop(0, DJ, mm, zeros16)
        plsc.store_scatter(trows, [jnp.zeros((L,), i32), iota * L + dl],
                           _tanh(acc))
    pltpu.sync_copy(trows.at[0], sh_slab.at[0, sid])
    plsc.subcore_barrier()
    pltpu.sync_copy(sh_slab.at[0], ttw)
    plsc.subcore_barrier()
    def t_build(w, _):
        for b in range(B):
            t_rm[b, pl.ds(w * L, L)] = ttw[w, pl.ds(b * L, L)]
        return 0
    lax.fori_loop(0, NS, t_build, 0)

    def p2_chunk(c, xbuf):
        def grp(g, _):
            bv = bidx2[pl.ds(c * CHUNK + g * L, L)]
            mn = lax.reduce_min(bv, (0,))
            mx = lax.reduce_max(bv, (0,))
            zt = (zeros16,) * L

            def uniform_dots():
                def jcl(jc, ps):
                    tv = t_rm[mn, pl.ds(jc * L, L)]
                    return tuple(
                        ps[r] + xbuf[g * L + r, pl.ds(jc * L, L)] * tv
                        for r in range(L))
                return lax.fori_loop(0, DJ, jcl, zt)

            def general_dots():
                def jcl(jc, ps):
                    out = []
                    for r in range(L):
                        tvr = t_rm[bv[r], pl.ds(jc * L, L)]
                        out.append(
                            ps[r] + xbuf[g * L + r, pl.ds(jc * L, L)] * tvr)
                    return tuple(out)
                return lax.fori_loop(0, DJ, jcl, zt)

            psums = lax.cond(mn == mx, uniform_dots, general_dots)
            dots = zeros16
            for r in range(L):
                dr = lax.reduce_sum(psums[r], (0,))
                dots = jnp.where(iota == r, dr, dots)
            coef = _sigmoid(dots)

            def uniform():
                def col(j, _):
                    s = zeros16
                    for r in range(L):
                        s = s + coef[r] * xbuf[g * L + r, pl.ds(j * L, L)]
                    out_l[mn, pl.ds(j * L, L)] += s
                    return 0
                lax.fori_loop(0, DJ, col, 0)

            def general():
                for r in range(L):
                    br = bv[r]
                    cr = coef[r]
                    def col2(j, _):
                        out_l[br, pl.ds(j * L, L)] += \
                            cr * xbuf[g * L + r, pl.ds(j * L, L)]
                        return 0
                    lax.fori_loop(0, DJ, col2, 0)

            lax.cond(mn == mx, uniform, general)
            return 0
        lax.fori_loop(0, GRP, grp, 0)

    def pair2(c2, _):
        for b in range(2):
            c = 2 * c2 + b
            pltpu.make_async_copy(
                x_hbm.at[pl.ds(base2, CHUNK)], bufs[b], sems[b]).wait()
            p2_chunk(c, bufs[b])

            @pl.when(c + 2 < NCHUNK2)
            def _pf():
                pltpu.async_copy(
                    x_hbm.at[pl.ds(base2 + (c + 2) * CHUNK, CHUNK)],
                    bufs[b], sems[b])
        return 0
    lax.fori_loop(0, NCHUNK2 // 2, pair2, 0)

    for h in range(2):
        for b8 in range(B // 2):
            pltpu.sync_copy(out_l.at[h * (B // 2) + b8], sh_slab.at[b8, sid])
        plsc.subcore_barrier()

        @pl.when(sid // (B // 2) == h)
        def _red2():
            pltpu.sync_copy(sh_slab.at[sid % (B // 2)], slabv)
            def red2(j, _):
                s = zeros16
                for r in range(NS):
                    s = s + slabv[r, pl.ds(j * L, L)]
                redrow[pl.ds(j * L, L)] = s
                return 0
            lax.fori_loop(0, DJ, red2, 0)
            pltpu.sync_copy(redrow, po_hbm.at[cid * B + sid])
        plsc.subcore_barrier()


def _bodyC(po_hbm, out_hbm, cbuf):
    cid = lax.axis_index("c")
    sid = lax.axis_index("s")

    @pl.when(cid == 0)
    def _c():
        pltpu.sync_copy(po_hbm.at[sid], cbuf.at[0])
        pltpu.sync_copy(po_hbm.at[B + sid], cbuf.at[1])
        def jc(j, _):
            cbuf[0, pl.ds(j * L, L)] += cbuf[1, pl.ds(j * L, L)]
            return 0
        lax.fori_loop(0, DJ, jc, 0)
        pltpu.sync_copy(cbuf.at[0], out_hbm.at[sid])


@jax.jit
def kernel(x, batch, W):
    wt = W.T
    mesh = plsc.VectorSubcoreMesh(core_axis_name="c", subcore_axis_name="s")
    cp = pltpu.CompilerParams(needs_layout_passes=False)

    runAB = pl.kernel(
        _bodyAB,
        out_type=jax.ShapeDtypeStruct((NC * B, D), f32),
        mesh=mesh, compiler_params=cp,
        scratch_types=[
            pltpu.VMEM((CHUNK, D), f32),
            pltpu.VMEM((CHUNK, D), f32),
            pltpu.VMEM((ROWS_P1,), i32),
            pltpu.VMEM((ROWS_P2,), i32),
            pltpu.VMEM((B, D), f32),
            pltpu.VMEM((B, D), f32),
            pltpu.VMEM((1, D), f32),
            pltpu.VMEM((NS, D), f32),
            pltpu.VMEM((D,), f32),
            pltpu.VMEM((NS, D), f32),
            pltpu.VMEM((B, D), f32),
            pltpu.VMEM((D, B), f32),
            pltpu.VMEM((DL_PER_W, D), f32),
            pltpu.VMEM((1, D), f32),
            pltpu.VMEM((NS, D), f32),
            pltpu.VMEM((B, D), f32),
            pltpu.SemaphoreType.DMA,
            pltpu.SemaphoreType.DMA,
            pltpu.VMEM_SHARED((B // 2, NS, D), f32),
            pltpu.VMEM_SHARED((NS, D), f32),
            pltpu.VMEM_SHARED((B, D), f32),
        ],
    )
    po = runAB(x, batch, wt)

    runC = pl.kernel(
        _bodyC,
        out_type=jax.ShapeDtypeStruct((B, D), f32),
        mesh=mesh, compiler_params=cp,
        scratch_types=[
            pltpu.VMEM((2, D), f32),
        ],
    )
    return runC(po)
